# baseline (speedup 1.0000x reference)
import functools

import jax
import jax.numpy as jnp
from jax import lax
from jax.experimental import pallas as pl
from jax.experimental.pallas import tpu as pltpu
from jax.experimental.pallas import tpu_sc as plsc

PROG = (3, 5, 7, 1, 4, 2, 6, 0)

_NC = 2
_NS = 16
_NW = _NC * _NS


def _chunk_width(per_w: int) -> int:
    for cw in range(128, 0, -8):
        if per_w % cw == 0:
            return cw
    raise ValueError(f"no valid chunk width for {per_w}")


def _make_gather(rows: int, hid: int):
    per_w = rows // _NW
    cw = _chunk_width(per_w)
    chunks = per_w // cw

    mesh = plsc.VectorSubcoreMesh(
        core_axis_name="c", subcore_axis_name="s", num_cores=_NC
    )

    @functools.partial(
        pl.kernel,
        mesh=mesh,
        out_type=jax.ShapeDtypeStruct((_NW, chunks, cw, hid), jnp.float32),
        scratch_types=[
            pltpu.VMEM((chunks, cw), jnp.int32),
            pltpu.VMEM((cw, hid), jnp.float32),
            pltpu.SemaphoreType.DMA,
        ],
    )
    def gather(tok_hbm, embed_hbm, out_hbm, idx_v, rows_v, sem):
        wid = lax.axis_index("s") * _NC + lax.axis_index("c")
        pltpu.sync_copy(tok_hbm.at[wid], idx_v)

        def body(j, carry):
            pltpu.async_copy(embed_hbm.at[idx_v.at[j]], rows_v, sem).wait()
            pltpu.sync_copy(rows_v, out_hbm.at[wid, j])
            return carry

        lax.fori_loop(0, chunks, body, 0, unroll=False)

    return gather, (chunks, cw)


def _logits_body(out_ref, pat_ref):
    bb, t, v = out_ref.shape

    @pl.when(pl.program_id(0) == 0)
    def _():
        ti = lax.broadcasted_iota(jnp.int32, (t, v), 0)
        vi = lax.broadcasted_iota(jnp.int32, (t, v), 1)
        tm = lax.rem(ti, 8)
        tgt = jnp.full((t, v), PROG[0], jnp.int32)
        for k in range(1, 8):
            tgt = jnp.where(tm == k, PROG[k], tgt)
        pat_ref[...] = jnp.where(vi == tgt, 1.0, -1000000000.0).astype(jnp.float32)

    out_ref[...] = jnp.broadcast_to(pat_ref[...][None], (bb, t, v))


def _make_logits(b: int, t: int, v: int, bb: int = 32):
    return pl.pallas_call(
        _logits_body,
        grid=(b // bb,),
        out_specs=pl.BlockSpec((bb, t, v), lambda i: (i, 0, 0)),
        out_shape=jax.ShapeDtypeStruct((b, t, v), jnp.float32),
        scratch_shapes=[pltpu.VMEM((t, v), jnp.float32)],
    )


def kernel(tokens, embed):
    b, t = tokens.shape
    vocab, hid = embed.shape
    rows = b * t

    gather, (chunks, cw) = _make_gather(rows, hid)
    tok = tokens.astype(jnp.int32).reshape(_NW, chunks, cw)
    h = gather(tok, embed).reshape(b, t, hid)

    logits = _make_logits(b, t, vocab)()
    return h, logits


# speedup vs baseline: 3.5896x; 3.5896x over previous
import functools

import jax
import jax.numpy as jnp
from jax import lax
from jax.experimental import pallas as pl
from jax.experimental.pallas import tpu as pltpu
from jax.experimental.pallas import tpu_sc as plsc

PROG = (3, 5, 7, 1, 4, 2, 6, 0)

_NC = 2
_NS = 16
_NW = _NC * _NS


def _chunk_width(per_w: int) -> int:
    for cw in range(128, 0, -8):
        if per_w % cw == 0:
            return cw
    raise ValueError(f"no valid chunk width for {per_w}")


def _make_gather(rows: int, hid: int):
    per_w = rows // _NW
    cw = _chunk_width(per_w)
    chunks = per_w // cw

    mesh = plsc.VectorSubcoreMesh(
        core_axis_name="c", subcore_axis_name="s", num_cores=_NC
    )

    @functools.partial(
        pl.kernel,
        mesh=mesh,
        out_type=jax.ShapeDtypeStruct((_NW, chunks, cw, hid), jnp.float32),
        scratch_types=[
            pltpu.VMEM((chunks, cw), jnp.int32),
            pltpu.VMEM((cw, hid), jnp.float32),
            pltpu.SemaphoreType.DMA,
        ],
    )
    def gather(tok_hbm, embed_hbm, out_hbm, idx_v, rows_v, sem):
        wid = lax.axis_index("s") * _NC + lax.axis_index("c")
        pltpu.sync_copy(tok_hbm.at[wid], idx_v)

        def body(j, carry):
            pltpu.async_copy(embed_hbm.at[idx_v.at[j]], rows_v, sem).wait()
            pltpu.sync_copy(rows_v, out_hbm.at[wid, j])
            return carry

        lax.fori_loop(0, chunks, body, 0, unroll=False)

    return gather, (chunks, cw)


def _logits_body(out_ref):
    _, v, b = out_ref.shape
    t = pl.program_id(0)
    tm = lax.rem(t, 8)
    tgt = jnp.int32(PROG[0])
    for k in range(1, 8):
        tgt = jnp.where(tm == k, jnp.int32(PROG[k]), tgt)
    vi = lax.broadcasted_iota(jnp.int32, (1, v, b), 1)
    out_ref[...] = jnp.where(vi == tgt, 1.0, -1000000000.0).astype(jnp.float32)


def _make_logits(b: int, t: int, v: int):
    return pl.pallas_call(
        _logits_body,
        grid=(t,),
        out_specs=pl.BlockSpec((1, v, b), lambda i: (i, 0, 0)),
        out_shape=jax.ShapeDtypeStruct((t, v, b), jnp.float32),
    )


def kernel(tokens, embed):
    b, t = tokens.shape
    vocab, hid = embed.shape
    rows = b * t

    gather, (chunks, cw) = _make_gather(rows, hid)
    tok = tokens.astype(jnp.int32).T.reshape(_NW, chunks, cw)
    h = gather(tok, embed).reshape(t, b, hid).transpose(1, 0, 2)

    logits = _make_logits(b, t, vocab)().transpose(2, 0, 1)
    return h, logits


# speedup vs baseline: 4.4618x; 1.2430x over previous
import functools

import jax
import jax.numpy as jnp
from jax import lax
from jax.experimental import pallas as pl
from jax.experimental.pallas import tpu as pltpu
from jax.experimental.pallas import tpu_sc as plsc

PROG = (3, 5, 7, 1, 4, 2, 6, 0)

_NC = 2
_NS = 16
_NW = _NC * _NS


def _chunk_width(per_w: int) -> int:
    for cw in range(128, 0, -8):
        if per_w % cw == 0:
            return cw
    raise ValueError(f"no valid chunk width for {per_w}")


def _make_gather(rows: int, vocab: int, hid: int):
    per_w = rows // _NW
    cw = _chunk_width(per_w)
    chunks = per_w // cw

    mesh = plsc.VectorSubcoreMesh(
        core_axis_name="c", subcore_axis_name="s", num_cores=_NC
    )

    @functools.partial(
        pl.kernel,
        mesh=mesh,
        out_type=jax.ShapeDtypeStruct((_NW, chunks, cw, hid), jnp.float32),
        scratch_types=[
            pltpu.VMEM((chunks, cw), jnp.int32),
            pltpu.VMEM((cw, hid), jnp.float32),
            pltpu.VMEM_SHARED((vocab, hid), jnp.float32),
            pltpu.SemaphoreType.DMA,
        ],
    )
    def gather(tok_hbm, embed_hbm, out_hbm, idx_v, rows_v, emb_s, sem):
        sid = lax.axis_index("s")
        wid = sid * _NC + lax.axis_index("c")
        @pl.when(sid == 0)
        def _():
            pltpu.sync_copy(embed_hbm, emb_s)

        pltpu.sync_copy(tok_hbm.at[wid], idx_v)
        plsc.subcore_barrier()

        def body(j, carry):
            pltpu.async_copy(emb_s.at[idx_v.at[j]], rows_v, sem).wait()
            pltpu.sync_copy(rows_v, out_hbm.at[wid, j])
            return carry

        lax.fori_loop(0, chunks, body, 0, unroll=False)

    return gather, (chunks, cw)


def _logits_body(out_ref):
    _, v, b = out_ref.shape
    t = pl.program_id(0)
    tm = lax.rem(t, 8)
    tgt = jnp.int32(PROG[0])
    for k in range(1, 8):
        tgt = jnp.where(tm == k, jnp.int32(PROG[k]), tgt)
    vi = lax.broadcasted_iota(jnp.int32, (1, v, b), 1)
    out_ref[...] = jnp.where(vi == tgt, 1.0, -1000000000.0).astype(jnp.float32)


def _make_logits(b: int, t: int, v: int):
    return pl.pallas_call(
        _logits_body,
        grid=(t,),
        out_specs=pl.BlockSpec((1, v, b), lambda i: (i, 0, 0)),
        out_shape=jax.ShapeDtypeStruct((t, v, b), jnp.float32),
    )


def kernel(tokens, embed):
    b, t = tokens.shape
    vocab, hid = embed.shape
    rows = b * t

    gather, (chunks, cw) = _make_gather(rows, vocab, hid)
    tok = tokens.astype(jnp.int32).T.reshape(_NW, chunks, cw)
    h = gather(tok, embed).reshape(t, b, hid).transpose(1, 0, 2)

    logits = _make_logits(b, t, vocab)().transpose(2, 0, 1)
    return h, logits
